# baseline (speedup 1.0000x reference)
import functools

import jax
import jax.numpy as jnp
from jax import lax
from jax.experimental import pallas as pl
from jax.experimental.pallas import tpu as pltpu
from jax.experimental.pallas import tpu_sc as plsc

N_NODES = 10000
E1 = 320000
E2 = 640000
H = 128

NC = 2
NS = 16
NW = NC * NS
EPW = E1 // NW
R = E2 // NW

_mesh = plsc.VectorSubcoreMesh(core_axis_name="c", subcore_axis_name="s")
_sc_params = pltpu.CompilerParams(needs_layout_passes=False)


def _worker_id():
    return lax.axis_index("s") * NC + lax.axis_index("c")


CG = 80
NCH = EPW // CG


def _gather_body(h_hbm, te_hbm, src_hbm, dst_hbm, e2_hbm,
                 hs_hbm, hd_hbm, st_hbm,
                 isrc, idst, ie2,
                 bs0, bd0, bt0, bs1, bd1, bt1,
                 sg0, sg1, so0, so1):
    base = _worker_id() * EPW
    bufs = ((bs0, bd0, bt0), (bs1, bd1, bt1))
    gsems = (sg0, sg1)
    osems = (so0, so1)

    pltpu.sync_copy(src_hbm.at[pl.ds(base, EPW)], isrc)
    pltpu.sync_copy(dst_hbm.at[pl.ds(base, EPW)], idst)
    pltpu.sync_copy(e2_hbm.at[pl.ds(base, EPW)], ie2)

    def issue_gather(k, slot):
        bs, bd, bt = bufs[slot]
        pltpu.async_copy(h_hbm.at[isrc.at[pl.ds(k * CG, CG)]], bs, gsems[slot])
        pltpu.async_copy(h_hbm.at[idst.at[pl.ds(k * CG, CG)]], bd, gsems[slot])
        pltpu.async_copy(te_hbm.at[ie2.at[pl.ds(k * CG, CG)]], bt, gsems[slot])

    def wait_gather(slot):
        bs, bd, bt = bufs[slot]
        pltpu.make_async_copy(h_hbm.at[isrc.at[pl.ds(0, CG)]], bs, gsems[slot]).wait()
        pltpu.make_async_copy(h_hbm.at[idst.at[pl.ds(0, CG)]], bd, gsems[slot]).wait()
        pltpu.make_async_copy(te_hbm.at[ie2.at[pl.ds(0, CG)]], bt, gsems[slot]).wait()

    def issue_out(k, slot):
        bs, bd, bt = bufs[slot]
        off = base + k * CG
        pltpu.async_copy(bs, hs_hbm.at[pl.ds(off, CG)], osems[slot])
        pltpu.async_copy(bd, hd_hbm.at[pl.ds(off, CG)], osems[slot])
        pltpu.async_copy(bt, st_hbm.at[pl.ds(off, CG)], osems[slot])

    def wait_out(slot):
        bs, bd, bt = bufs[slot]
        off = base
        pltpu.make_async_copy(bs, hs_hbm.at[pl.ds(off, CG)], osems[slot]).wait()
        pltpu.make_async_copy(bd, hd_hbm.at[pl.ds(off, CG)], osems[slot]).wait()
        pltpu.make_async_copy(bt, st_hbm.at[pl.ds(off, CG)], osems[slot]).wait()

    issue_gather(0, 0)
    issue_gather(1, 1)

    def ring(i, carry):
        k0 = 2 * i
        wait_gather(0)
        issue_out(k0, 0)
        wait_gather(1)
        issue_out(k0 + 1, 1)
        wait_out(0)
        issue_gather(k0 + 2, 0)
        wait_out(1)

        @pl.when(i < (NCH - 1) // 2 - 1)
        def _():
            issue_gather(k0 + 3, 1)
        return carry

    lax.fori_loop(0, (NCH - 1) // 2, ring, 0)
    wait_gather(0)
    issue_out(NCH - 1, 0)
    wait_out(0)


_gather_call = pl.kernel(
    _gather_body,
    out_type=(
        jax.ShapeDtypeStruct((E1, H), jnp.float32),
        jax.ShapeDtypeStruct((E1, H), jnp.float32),
        jax.ShapeDtypeStruct((E1, H), jnp.float32),
    ),
    mesh=_mesh,
    scratch_types=[
        pltpu.VMEM((EPW,), jnp.int32),
        pltpu.VMEM((EPW,), jnp.int32),
        pltpu.VMEM((EPW,), jnp.int32),
        pltpu.VMEM((CG, H), jnp.float32),
        pltpu.VMEM((CG, H), jnp.float32),
        pltpu.VMEM((CG, H), jnp.float32),
        pltpu.VMEM((CG, H), jnp.float32),
        pltpu.VMEM((CG, H), jnp.float32),
        pltpu.VMEM((CG, H), jnp.float32),
        pltpu.SemaphoreType.DMA,
        pltpu.SemaphoreType.DMA,
        pltpu.SemaphoreType.DMA,
        pltpu.SemaphoreType.DMA,
    ],
    compiler_params=_sc_params,
)


BLK = 3200
NB_MLP = E1 // BLK


def _mlp_body(hs, hd, st, w1a, w1b, w1c, w2, b1, b2, out):
    x = (jnp.dot(hs[...], w1a[...], preferred_element_type=jnp.float32)
         + jnp.dot(hd[...], w1b[...], preferred_element_type=jnp.float32)
         + jnp.dot(st[...], w1c[...], preferred_element_type=jnp.float32)
         + b1[...])
    hid = x * jax.nn.sigmoid(x)
    out[...] = (st[...] + b2[...]
                + jnp.dot(hid, w2[...], preferred_element_type=jnp.float32))


def _mlp_v(hs, hd, st, w1a, w1b, w1c, w2, b1, b2):
    return pl.pallas_call(
        _mlp_body,
        grid=(NB_MLP,),
        in_specs=[
            pl.BlockSpec((BLK, H), lambda g: (g, 0)),
            pl.BlockSpec((BLK, H), lambda g: (g, 0)),
            pl.BlockSpec((BLK, H), lambda g: (g, 0)),
            pl.BlockSpec((H, H), lambda g: (0, 0)),
            pl.BlockSpec((H, H), lambda g: (0, 0)),
            pl.BlockSpec((H, H), lambda g: (0, 0)),
            pl.BlockSpec((H, H), lambda g: (0, 0)),
            pl.BlockSpec((1, H), lambda g: (0, 0)),
            pl.BlockSpec((1, H), lambda g: (0, 0)),
        ],
        out_specs=pl.BlockSpec((BLK, H), lambda g: (g, 0)),
        out_shape=jax.ShapeDtypeStruct((E1, H), jnp.float32),
    )(hs, hd, st, w1a, w1b, w1c, w2, b1, b2)


CI = 2000
NVI = CI // 16
UNR = 5
NCI = E1 // CI


def _scan_body(e2_hbm, te_hbm, o_hbm, wtab_hbm, table, ib0, ib1, si0, si1, scp):
    sbase = _worker_id() * R
    ibufs = (ib0, ib1)
    isems = (si0, si1)

    cp = pltpu.async_copy(te_hbm.at[pl.ds(sbase, R)],
                          o_hbm.at[pl.ds(sbase, R)], scp)

    def initb(i, carry):
        table[pl.ds(i * 16, 16)] = jnp.full((16,), -1, jnp.int32)
        return carry

    lax.fori_loop(0, R // 16, initb, 0)

    def issue_idx(k, slot):
        pltpu.async_copy(e2_hbm.at[pl.ds(k * CI, CI)], ibufs[slot], isems[slot])

    def wait_idx(slot):
        pltpu.make_async_copy(e2_hbm.at[pl.ds(0, CI)], ibufs[slot], isems[slot]).wait()

    def scan_chunk(k, slot):
        ibuf = ibufs[slot]

        def inner(i, evec):
            locs, ms, es = [], [], []
            for u in range(UNR):
                s = ibuf[pl.ds(i * (16 * UNR) + u * 16, 16)]
                loc = s - sbase
                locs.append(loc)
                ms.append(plsc.bitcast(loc, jnp.uint32) < jnp.uint32(R))
                es.append(evec + (u * 16))
            for u in range(UNR):
                plsc.store_scatter(table, [locs[u]], es[u], mask=ms[u])
            curs = [plsc.load_gather(table, [locs[u]], mask=ms[u])
                    for u in range(UNR)]
            for u in range(UNR):
                m2 = ms[u] & (es[u] > curs[u])
                plsc.store_scatter(table, [locs[u]], es[u], mask=m2)
            return evec + (16 * UNR)

        lax.fori_loop(0, NVI // UNR, inner,
                      lax.iota(jnp.int32, 16) + (k * CI))

    issue_idx(0, 0)
    issue_idx(1, 1)

    def scanring(i, carry):
        k0 = 2 * i
        wait_idx(0)
        scan_chunk(k0, 0)

        @pl.when(i < NCI // 2 - 1)
        def _():
            issue_idx(k0 + 2, 0)

        wait_idx(1)
        scan_chunk(k0 + 1, 1)

        @pl.when(i < NCI // 2 - 1)
        def _():
            issue_idx(k0 + 3, 1)
        return carry

    lax.fori_loop(0, NCI // 2, scanring, 0)
    pltpu.sync_copy(table, wtab_hbm.at[pl.ds(sbase, R)])
    cp.wait()


_scan_call = pl.kernel(
    _scan_body,
    out_type=jax.ShapeDtypeStruct((E2,), jnp.int32),
    mesh=_mesh,
    scratch_types=[
        pltpu.VMEM((R,), jnp.int32),
        pltpu.VMEM((CI,), jnp.int32),
        pltpu.VMEM((CI,), jnp.int32),
        pltpu.SemaphoreType.DMA,
        pltpu.SemaphoreType.DMA,
        pltpu.SemaphoreType.DMA,
    ],
    compiler_params=_sc_params,
)


GCH = 128
NCW = (R + GCH - 1) // GCH + 1


def _winner_body(wtab_hbm, v_hbm, o_hbm,
                 tbuf, ws2, we2, rb0, rb1,
                 sg0, sg1, ss0, ss1):
    sbase = _worker_id() * R
    rbufs = (rb0, rb1)
    gsems = (sg0, sg1)
    ssems = (ss0, ss1)

    pltpu.sync_copy(wtab_hbm.at[pl.ds(sbase, R)], tbuf)

    def comp(i, nwv):
        t = tbuf[pl.ds(i * 16, 16)]
        m = t >= 0
        slots = lax.iota(jnp.int32, 16) + (sbase + i * 16)
        cnt = plsc.cumsum(jnp.where(m, 1, 0))
        pos = nwv + cnt - 1
        plsc.store_scatter(ws2, [pos >> 7, pos & 127], slots, mask=m)
        plsc.store_scatter(we2, [pos >> 7, pos & 127], t, mask=m)
        return nwv + plsc.all_reduce_population_count(m)

    nwv = lax.fori_loop(0, R // 16, comp,
                        jnp.zeros((16,), jnp.int32))

    z = jnp.zeros((16,), jnp.int32)
    w0s = plsc.load_gather(ws2, [z, z])
    w0e = plsc.load_gather(we2, [z, z])
    end = ((nwv + 127) >> 7) << 7
    for j in range(GCH // 16):
        pos = nwv + lax.iota(jnp.int32, 16) + (j * 16)
        mf = pos < end
        plsc.store_scatter(ws2, [pos >> 7, pos & 127], w0s, mask=mf)
        plsc.store_scatter(we2, [pos >> 7, pos & 127], w0e, mask=mf)

    nw = jnp.max(nwv)
    nch = (nw + GCH - 1) // GCH

    def issue_gather(c, slot):
        pltpu.async_copy(v_hbm.at[we2.at[c]], rbufs[slot], gsems[slot])

    def wait_gather(slot):
        pltpu.make_async_copy(v_hbm.at[we2.at[0]], rbufs[slot], gsems[slot]).wait()

    def issue_scat(c, slot):
        pltpu.async_copy(rbufs[slot], o_hbm.at[ws2.at[c]], ssems[slot])

    def wait_scat(slot):
        pltpu.make_async_copy(rbufs[slot], o_hbm.at[ws2.at[0]], ssems[slot]).wait()

    def pair(i, carry):
        c0 = 2 * i
        c1 = c0 + 1
        issue_gather(c0, 0)

        @pl.when(c1 < nch)
        def _():
            issue_gather(c1, 1)

        wait_gather(0)
        issue_scat(c0, 0)

        @pl.when(c1 < nch)
        def _():
            wait_gather(1)
            issue_scat(c1, 1)

        wait_scat(0)

        @pl.when(c1 < nch)
        def _():
            wait_scat(1)
        return carry

    lax.fori_loop(0, (nch + 1) // 2, pair, 0)


_winner_call = pl.kernel(
    _winner_body,
    out_type=(),
    mesh=_mesh,
    scratch_types=[
        pltpu.VMEM((R,), jnp.int32),
        pltpu.VMEM((NCW, GCH), jnp.int32),
        pltpu.VMEM((NCW, GCH), jnp.int32),
        pltpu.VMEM((GCH, H), jnp.float32),
        pltpu.VMEM((GCH, H), jnp.float32),
        pltpu.SemaphoreType.DMA,
        pltpu.SemaphoreType.DMA,
        pltpu.SemaphoreType.DMA,
        pltpu.SemaphoreType.DMA,
    ],
    compiler_params=_sc_params,
)


def kernel(t_e2, h, edge_index1, e1_to_e2, W1, b1, W2, b2):
    src = edge_index1[0].astype(jnp.int32)
    dst = edge_index1[1].astype(jnp.int32)
    e2i = e1_to_e2.astype(jnp.int32)
    o_ref = jax.new_ref(jax.lax.empty((E2, H), jnp.float32))
    hs, hd, st = _gather_call(h, t_e2, src, dst, e2i)
    wtab = _scan_call(e2i, t_e2, o_ref)
    v = _mlp_v(hs, hd, st,
               W1[0:H], W1[H:2 * H], W1[2 * H:3 * H], W2,
               b1.reshape(1, H), b2.reshape(1, H))
    _winner_call(wtab, v, o_ref)
    return jax.freeze(o_ref)


# speedup vs baseline: 10.7413x; 10.7413x over previous
import functools

import jax
import jax.numpy as jnp
from jax import lax
from jax.experimental import pallas as pl
from jax.experimental.pallas import tpu as pltpu
from jax.experimental.pallas import tpu_sc as plsc

N_NODES = 10000
E1 = 320000
E2 = 640000
H = 128

NC = 2
NS = 16
NW = NC * NS
EPW = E1 // NW
R = E2 // NW

_mesh = plsc.VectorSubcoreMesh(core_axis_name="c", subcore_axis_name="s")
_sc_params = pltpu.CompilerParams(needs_layout_passes=False)


def _worker_id():
    return lax.axis_index("s") * NC + lax.axis_index("c")


CG = 80
NCH = EPW // CG


def _gather_body(h_hbm, te_hbm, src_hbm, dst_hbm, e2_hbm,
                 hs_hbm, hd_hbm, st_hbm,
                 isrc, idst, ie2,
                 bs0, bd0, bt0, bs1, bd1, bt1,
                 sg0, sg1, so0, so1):
    base = _worker_id() * EPW
    bufs = ((bs0, bd0, bt0), (bs1, bd1, bt1))
    gsems = (sg0, sg1)
    osems = (so0, so1)

    pltpu.sync_copy(src_hbm.at[pl.ds(base, EPW)], isrc)
    pltpu.sync_copy(dst_hbm.at[pl.ds(base, EPW)], idst)
    pltpu.sync_copy(e2_hbm.at[pl.ds(base, EPW)], ie2)

    def issue_gather(k, slot):
        bs, bd, bt = bufs[slot]
        pltpu.async_copy(h_hbm.at[isrc.at[pl.ds(k * CG, CG)]], bs, gsems[slot])
        pltpu.async_copy(h_hbm.at[idst.at[pl.ds(k * CG, CG)]], bd, gsems[slot])
        pltpu.async_copy(te_hbm.at[ie2.at[pl.ds(k * CG, CG)]], bt, gsems[slot])

    def wait_gather(slot):
        bs, bd, bt = bufs[slot]
        pltpu.make_async_copy(h_hbm.at[isrc.at[pl.ds(0, CG)]], bs, gsems[slot]).wait()
        pltpu.make_async_copy(h_hbm.at[idst.at[pl.ds(0, CG)]], bd, gsems[slot]).wait()
        pltpu.make_async_copy(te_hbm.at[ie2.at[pl.ds(0, CG)]], bt, gsems[slot]).wait()

    def issue_out(k, slot):
        bs, bd, bt = bufs[slot]
        off = base + k * CG
        pltpu.async_copy(bs, hs_hbm.at[pl.ds(off, CG)], osems[slot])
        pltpu.async_copy(bd, hd_hbm.at[pl.ds(off, CG)], osems[slot])
        pltpu.async_copy(bt, st_hbm.at[pl.ds(off, CG)], osems[slot])

    def wait_out(slot):
        bs, bd, bt = bufs[slot]
        off = base
        pltpu.make_async_copy(bs, hs_hbm.at[pl.ds(off, CG)], osems[slot]).wait()
        pltpu.make_async_copy(bd, hd_hbm.at[pl.ds(off, CG)], osems[slot]).wait()
        pltpu.make_async_copy(bt, st_hbm.at[pl.ds(off, CG)], osems[slot]).wait()

    issue_gather(0, 0)
    issue_gather(1, 1)

    def ring(i, carry):
        k0 = 2 * i
        wait_gather(0)
        issue_out(k0, 0)
        wait_gather(1)
        issue_out(k0 + 1, 1)
        wait_out(0)
        issue_gather(k0 + 2, 0)
        wait_out(1)

        @pl.when(i < (NCH - 1) // 2 - 1)
        def _():
            issue_gather(k0 + 3, 1)
        return carry

    lax.fori_loop(0, (NCH - 1) // 2, ring, 0)
    wait_gather(0)
    issue_out(NCH - 1, 0)
    wait_out(0)


_gather_call = pl.kernel(
    _gather_body,
    out_type=(
        jax.ShapeDtypeStruct((E1, H), jnp.float32),
        jax.ShapeDtypeStruct((E1, H), jnp.float32),
        jax.ShapeDtypeStruct((E1, H), jnp.float32),
    ),
    mesh=_mesh,
    scratch_types=[
        pltpu.VMEM((EPW,), jnp.int32),
        pltpu.VMEM((EPW,), jnp.int32),
        pltpu.VMEM((EPW,), jnp.int32),
        pltpu.VMEM((CG, H), jnp.float32),
        pltpu.VMEM((CG, H), jnp.float32),
        pltpu.VMEM((CG, H), jnp.float32),
        pltpu.VMEM((CG, H), jnp.float32),
        pltpu.VMEM((CG, H), jnp.float32),
        pltpu.VMEM((CG, H), jnp.float32),
        pltpu.SemaphoreType.DMA,
        pltpu.SemaphoreType.DMA,
        pltpu.SemaphoreType.DMA,
        pltpu.SemaphoreType.DMA,
    ],
    compiler_params=_sc_params,
)


BLK = 3200
NB_MLP = E1 // BLK


def _mlp_body(hs, hd, st, w1a, w1b, w1c, w2, b1, b2, out):
    x = (jnp.dot(hs[...], w1a[...], preferred_element_type=jnp.float32)
         + jnp.dot(hd[...], w1b[...], preferred_element_type=jnp.float32)
         + jnp.dot(st[...], w1c[...], preferred_element_type=jnp.float32)
         + b1[...])
    hid = x * jax.nn.sigmoid(x)
    out[...] = (st[...] + b2[...]
                + jnp.dot(hid, w2[...], preferred_element_type=jnp.float32))


def _mlp_v(hs, hd, st, w1a, w1b, w1c, w2, b1, b2):
    return pl.pallas_call(
        _mlp_body,
        grid=(NB_MLP,),
        in_specs=[
            pl.BlockSpec((BLK, H), lambda g: (g, 0)),
            pl.BlockSpec((BLK, H), lambda g: (g, 0)),
            pl.BlockSpec((BLK, H), lambda g: (g, 0)),
            pl.BlockSpec((H, H), lambda g: (0, 0)),
            pl.BlockSpec((H, H), lambda g: (0, 0)),
            pl.BlockSpec((H, H), lambda g: (0, 0)),
            pl.BlockSpec((H, H), lambda g: (0, 0)),
            pl.BlockSpec((1, H), lambda g: (0, 0)),
            pl.BlockSpec((1, H), lambda g: (0, 0)),
        ],
        out_specs=pl.BlockSpec((BLK, H), lambda g: (g, 0)),
        out_shape=jax.ShapeDtypeStruct((E1, H), jnp.float32),
    )(hs, hd, st, w1a, w1b, w1c, w2, b1, b2)


CI = 2000
NVI = CI // 16
UNR = 5
NCI = E1 // CI


def _scan_body(e2_hbm, wtab_hbm, table, ib0, ib1, si0, si1):
    sbase = _worker_id() * R
    ibufs = (ib0, ib1)
    isems = (si0, si1)

    def initb(i, carry):
        table[pl.ds(i * 16, 16)] = jnp.full((16,), -1, jnp.int32)
        return carry

    lax.fori_loop(0, R // 16, initb, 0)

    def issue_idx(k, slot):
        pltpu.async_copy(e2_hbm.at[pl.ds(k * CI, CI)], ibufs[slot], isems[slot])

    def wait_idx(slot):
        pltpu.make_async_copy(e2_hbm.at[pl.ds(0, CI)], ibufs[slot], isems[slot]).wait()

    def scan_chunk(k, slot):
        ibuf = ibufs[slot]

        def inner(i, evec):
            locs, ms, es = [], [], []
            for u in range(UNR):
                s = ibuf[pl.ds(i * (16 * UNR) + u * 16, 16)]
                loc = s - sbase
                locs.append(loc)
                ms.append(plsc.bitcast(loc, jnp.uint32) < jnp.uint32(R))
                es.append(evec + (u * 16))
            for u in range(UNR):
                plsc.store_scatter(table, [locs[u]], es[u], mask=ms[u])
            curs = [plsc.load_gather(table, [locs[u]], mask=ms[u])
                    for u in range(UNR)]
            for u in range(UNR):
                m2 = ms[u] & (es[u] > curs[u])
                plsc.store_scatter(table, [locs[u]], es[u], mask=m2)
            return evec + (16 * UNR)

        lax.fori_loop(0, NVI // UNR, inner,
                      lax.iota(jnp.int32, 16) + (k * CI))

    issue_idx(0, 0)
    issue_idx(1, 1)

    def scanring(i, carry):
        k0 = 2 * i
        wait_idx(0)
        scan_chunk(k0, 0)

        @pl.when(i < NCI // 2 - 1)
        def _():
            issue_idx(k0 + 2, 0)

        wait_idx(1)
        scan_chunk(k0 + 1, 1)

        @pl.when(i < NCI // 2 - 1)
        def _():
            issue_idx(k0 + 3, 1)
        return carry

    lax.fori_loop(0, NCI // 2, scanring, 0)
    pltpu.sync_copy(table, wtab_hbm.at[pl.ds(sbase, R)])


_scan_call = pl.kernel(
    _scan_body,
    out_type=jax.ShapeDtypeStruct((E2,), jnp.int32),
    mesh=_mesh,
    scratch_types=[
        pltpu.VMEM((R,), jnp.int32),
        pltpu.VMEM((CI,), jnp.int32),
        pltpu.VMEM((CI,), jnp.int32),
        pltpu.SemaphoreType.DMA,
        pltpu.SemaphoreType.DMA,
    ],
    compiler_params=_sc_params,
)


CBLK = 3200
NB_CP = E2 // CBLK


def _copy_body(src, out):
    out[...] = src[...]


def _copy_te(t_e2):
    return pl.pallas_call(
        _copy_body,
        grid=(NB_CP,),
        in_specs=[pl.BlockSpec((CBLK, H), lambda g: (g, 0))],
        out_specs=pl.BlockSpec((CBLK, H), lambda g: (g, 0)),
        out_shape=jax.ShapeDtypeStruct((E2, H), jnp.float32),
    )(t_e2)


GCH = 128
NCW = (R + GCH - 1) // GCH + 1


def _winner_body(wtab_hbm, v_hbm, o_hbm,
                 tbuf, ws2, we2, rb0, rb1,
                 sg0, sg1, ss0, ss1):
    sbase = _worker_id() * R
    rbufs = (rb0, rb1)
    gsems = (sg0, sg1)
    ssems = (ss0, ss1)

    pltpu.sync_copy(wtab_hbm.at[pl.ds(sbase, R)], tbuf)

    def comp(i, nwv):
        t = tbuf[pl.ds(i * 16, 16)]
        m = t >= 0
        slots = lax.iota(jnp.int32, 16) + (sbase + i * 16)
        cnt = plsc.cumsum(jnp.where(m, 1, 0))
        pos = nwv + cnt - 1
        plsc.store_scatter(ws2, [pos >> 7, pos & 127], slots, mask=m)
        plsc.store_scatter(we2, [pos >> 7, pos & 127], t, mask=m)
        return nwv + plsc.all_reduce_population_count(m)

    nwv = lax.fori_loop(0, R // 16, comp,
                        jnp.zeros((16,), jnp.int32))

    z = jnp.zeros((16,), jnp.int32)
    w0s = plsc.load_gather(ws2, [z, z])
    w0e = plsc.load_gather(we2, [z, z])
    end = ((nwv + 127) >> 7) << 7
    for j in range(GCH // 16):
        pos = nwv + lax.iota(jnp.int32, 16) + (j * 16)
        mf = pos < end
        plsc.store_scatter(ws2, [pos >> 7, pos & 127], w0s, mask=mf)
        plsc.store_scatter(we2, [pos >> 7, pos & 127], w0e, mask=mf)

    nw = jnp.max(nwv)
    nch = (nw + GCH - 1) // GCH

    def issue_gather(c, slot):
        pltpu.async_copy(v_hbm.at[we2.at[c]], rbufs[slot], gsems[slot])

    def wait_gather(slot):
        pltpu.make_async_copy(v_hbm.at[we2.at[0]], rbufs[slot], gsems[slot]).wait()

    def issue_scat(c, slot):
        pltpu.async_copy(rbufs[slot], o_hbm.at[ws2.at[c]], ssems[slot])

    def wait_scat(slot):
        pltpu.make_async_copy(rbufs[slot], o_hbm.at[ws2.at[0]], ssems[slot]).wait()

    def pair(i, carry):
        c0 = 2 * i
        c1 = c0 + 1
        issue_gather(c0, 0)

        @pl.when(c1 < nch)
        def _():
            issue_gather(c1, 1)

        wait_gather(0)
        issue_scat(c0, 0)

        @pl.when(c1 < nch)
        def _():
            wait_gather(1)
            issue_scat(c1, 1)

        wait_scat(0)

        @pl.when(c1 < nch)
        def _():
            wait_scat(1)
        return carry

    lax.fori_loop(0, (nch + 1) // 2, pair, 0)


_winner_call = pl.kernel(
    _winner_body,
    out_type=(),
    mesh=_mesh,
    scratch_types=[
        pltpu.VMEM((R,), jnp.int32),
        pltpu.VMEM((NCW, GCH), jnp.int32),
        pltpu.VMEM((NCW, GCH), jnp.int32),
        pltpu.VMEM((GCH, H), jnp.float32),
        pltpu.VMEM((GCH, H), jnp.float32),
        pltpu.SemaphoreType.DMA,
        pltpu.SemaphoreType.DMA,
        pltpu.SemaphoreType.DMA,
        pltpu.SemaphoreType.DMA,
    ],
    compiler_params=_sc_params,
)


def kernel(t_e2, h, edge_index1, e1_to_e2, W1, b1, W2, b2):
    src = edge_index1[0].astype(jnp.int32)
    dst = edge_index1[1].astype(jnp.int32)
    e2i = e1_to_e2.astype(jnp.int32)
    hs, hd, st = _gather_call(h, t_e2, src, dst, e2i)
    wtab = _scan_call(e2i)
    ocopy = _copy_te(t_e2)
    v = _mlp_v(hs, hd, st,
               W1[0:H], W1[H:2 * H], W1[2 * H:3 * H], W2,
               b1.reshape(1, H), b2.reshape(1, H))
    o_ref = jax.new_ref(ocopy)
    _winner_call(wtab, v, o_ref)
    return jax.freeze(o_ref)
